# Initial kernel scaffold; baseline (speedup 1.0000x reference)
import jax
import jax.numpy as jnp
from jax.experimental import pallas as pl


def kernel(src, tgt_indices):
    raise NotImplementedError("write your pallas kernel here")


# speedup vs baseline: 2.9229x; 2.9229x over previous
import functools

import jax
import jax.numpy as jnp
from jax import lax
from jax.experimental import pallas as pl
from jax.experimental.pallas import tpu as pltpu
from jax.experimental.pallas import tpu_sc as plsc

_NC = 2
_NS = 16
_NW = _NC * _NS
_CH = 128

_CLIP_LO = 1e-8
_CLIP_HI = 1.0 - 1e-8


def _sc_gather_body(src_hbm, idx_hbm, out_hbm, idx_v, vals_v, sem):
    wid = lax.axis_index("s") * _NC + lax.axis_index("c")
    pltpu.sync_copy(idx_hbm.at[wid], idx_v)
    nch = idx_v.shape[0]
    copies = []
    for c in range(nch):
        copies.append(
            pltpu.async_copy(src_hbm.at[idx_v.at[c]], vals_v.at[c], sem)
        )
    for cp in copies:
        cp.wait()
    pltpu.sync_copy(vals_v, out_hbm.at[wid])


def _make_sc_gather(n_elems):
    assert n_elems % (_NW * _CH) == 0
    nch = n_elems // (_NW * _CH)
    return functools.partial(
        pl.kernel,
        out_type=jax.ShapeDtypeStruct((_NW, nch, _CH), jnp.float32),
        mesh=plsc.VectorSubcoreMesh(core_axis_name="c", subcore_axis_name="s"),
        scratch_types=[
            pltpu.VMEM((nch, _CH), jnp.int32),
            pltpu.VMEM((nch, _CH), jnp.float32),
            pltpu.SemaphoreType.DMA,
        ],
    )(_sc_gather_body)


def _tc_loss_body(src_ref, idx_ref, g_ref, out_ref, acc_ref):
    i = pl.program_id(0)
    n_i = pl.num_programs(0)

    p = jnp.clip(src_ref[...], _CLIP_LO, _CLIP_HI)
    dense = jnp.sum(jnp.log(1.0 - p))

    idx = idx_ref[...]
    g = jnp.clip(g_ref[...], _CLIP_LO, _CLIP_HI)
    r_blk, t = idx.shape
    col = lax.broadcasted_iota(jnp.int32, (r_blk, t), 1)
    dup = jnp.zeros((r_blk, t), jnp.bool_)
    for j in range(1, t):
        eq = (idx == idx[:, j:j + 1]) & (col < j)
        dup_j = jnp.any(eq, axis=1, keepdims=True)
        dup = dup | (dup_j & (col == j))
    corr_terms = jnp.where(dup, 0.0, jnp.log(1.0 - g) - jnp.log(g))
    corr = jnp.sum(corr_terms)

    @pl.when(i == 0)
    def _():
        acc_ref[0] = 0.0

    acc_ref[0] += corr - dense

    @pl.when(i == n_i - 1)
    def _():
        out_ref[0, 0] = acc_ref[0]


def _tc_loss(src, idx, g, block_rows=256):
    b, v = src.shape
    t = idx.shape[1]
    grid = (b // block_rows,)
    out = pl.pallas_call(
        _tc_loss_body,
        grid=grid,
        in_specs=[
            pl.BlockSpec((block_rows, v), lambda i: (i, 0)),
            pl.BlockSpec((block_rows, t), lambda i: (i, 0)),
            pl.BlockSpec((block_rows, t), lambda i: (i, 0)),
        ],
        out_specs=pl.BlockSpec(memory_space=pltpu.SMEM),
        out_shape=jax.ShapeDtypeStruct((1, 1), jnp.float32),
        scratch_shapes=[pltpu.SMEM((1,), jnp.float32)],
    )(src, idx, g)
    return out


def kernel(src, tgt_indices):
    b, v = src.shape
    t = tgt_indices.shape[1]
    idx32 = tgt_indices.astype(jnp.int32)
    flat_idx = idx32 + jnp.arange(b, dtype=jnp.int32)[:, None] * v
    n = b * t
    nch = n // (_NW * _CH)
    g = _make_sc_gather(n)(src.reshape(-1), flat_idx.reshape(_NW, nch, _CH))
    g = g.reshape(b, t)
    total = _tc_loss(src, idx32, g)
    scale = jnp.float32(1.0 / (b * v))
    return total[0, 0] * scale


# speedup vs baseline: 4.3662x; 1.4938x over previous
import functools

import jax
import jax.numpy as jnp
from jax import lax
from jax.experimental import pallas as pl
from jax.experimental.pallas import tpu as pltpu
from jax.experimental.pallas import tpu_sc as plsc

_NC = 2
_NS = 16
_NW = _NC * _NS
_CH = 128

_CLIP_LO = 1e-8
_CLIP_HI = 1.0 - 1e-8


def _sc_gather_body(src_hbm, idx_hbm, out_hbm, idx_v, vals_v, sem):
    wid = lax.axis_index("s") * _NC + lax.axis_index("c")
    pltpu.sync_copy(idx_hbm.at[wid], idx_v)
    nch = idx_v.shape[0]
    copies = []
    for c in range(nch):
        copies.append(
            pltpu.async_copy(src_hbm.at[idx_v.at[c]], vals_v.at[c], sem)
        )
    for cp in copies:
        cp.wait()
    pltpu.sync_copy(vals_v, out_hbm.at[wid])


def _make_sc_gather(n_elems):
    assert n_elems % (_NW * _CH) == 0
    nch = n_elems // (_NW * _CH)
    return functools.partial(
        pl.kernel,
        out_type=jax.ShapeDtypeStruct((_NW, nch, _CH), jnp.float32),
        mesh=plsc.VectorSubcoreMesh(core_axis_name="c", subcore_axis_name="s"),
        scratch_types=[
            pltpu.VMEM((nch, _CH), jnp.int32),
            pltpu.VMEM((nch, _CH), jnp.float32),
            pltpu.SemaphoreType.DMA,
        ],
    )(_sc_gather_body)


def _tc_loss_body(src_ref, idx_ref, g_ref, out_ref, acc_ref):
    i = pl.program_id(0)
    n_i = pl.num_programs(0)

    p = jnp.clip(src_ref[...], _CLIP_LO, _CLIP_HI)
    dense = jnp.sum(jnp.log(1.0 - p))

    @pl.when(i == 0)
    def _():
        acc_ref[0] = 0.0

    acc_ref[0] += -dense

    @pl.when(i == n_i - 1)
    def _():
        t = idx_ref.shape[0]
        tot = None
        for j in range(t):
            gj = jnp.clip(g_ref[j], _CLIP_LO, _CLIP_HI)
            fj = jnp.log(1.0 - gj) - jnp.log(gj)
            if j == 0:
                tot = fj
            else:
                ij = idx_ref[j]
                dup = ij == idx_ref[0]
                for k in range(1, j):
                    dup = dup | (ij == idx_ref[k])
                tot = tot + jnp.where(dup, 0.0, fj)
        out_ref[0, 0] = acc_ref[0] + jnp.sum(tot)


def _tc_loss(src, idx_t, g_t, block_rows=256):
    b, v = src.shape
    t, sub, lanes = idx_t.shape
    grid = (b // block_rows,)
    out = pl.pallas_call(
        _tc_loss_body,
        grid=grid,
        in_specs=[
            pl.BlockSpec((block_rows, v), lambda i: (i, 0)),
            pl.BlockSpec((t, sub, lanes), lambda i: (0, 0, 0)),
            pl.BlockSpec((t, sub, lanes), lambda i: (0, 0, 0)),
        ],
        out_specs=pl.BlockSpec(memory_space=pltpu.SMEM),
        out_shape=jax.ShapeDtypeStruct((1, 1), jnp.float32),
        scratch_shapes=[pltpu.SMEM((1,), jnp.float32)],
    )(src, idx_t, g_t)
    return out


def kernel(src, tgt_indices):
    b, v = src.shape
    t = tgt_indices.shape[1]
    idx32 = tgt_indices.astype(jnp.int32)
    idx_t = idx32.T.reshape(t, b // 128, 128)
    flat_idx_t = (idx32 + jnp.arange(b, dtype=jnp.int32)[:, None] * v).T
    n = b * t
    nch = n // (_NW * _CH)
    g_t = _make_sc_gather(n)(src.reshape(-1), flat_idx_t.reshape(_NW, nch, _CH))
    g_t = g_t.reshape(t, b // 128, 128)
    total = _tc_loss(src, idx_t, g_t)
    scale = jnp.float32(1.0 / (b * v))
    return total[0, 0] * scale


# speedup vs baseline: 5.0705x; 1.1613x over previous
import functools

import jax
import jax.numpy as jnp
from jax import lax
from jax.experimental import pallas as pl
from jax.experimental.pallas import tpu as pltpu
from jax.experimental.pallas import tpu_sc as plsc

_NC = 2
_NS = 16
_NW = _NC * _NS
_CH = 128

_CLIP_LO = 1e-8
_CLIP_HI = 1.0 - 1e-8
_LANES = 128


def _sc_gather_body(src_hbm, idx_hbm, out_hbm, idx_v, vals_v, sem):
    wid = lax.axis_index("s") * _NC + lax.axis_index("c")
    nch = idx_v.shape[0]
    rows = pl.ds(wid * nch, nch)
    pltpu.sync_copy(idx_hbm.at[rows], idx_v)
    copies = []
    for c in range(nch):
        copies.append(
            pltpu.async_copy(src_hbm.at[idx_v.at[c]], vals_v.at[c], sem)
        )
    for cp in copies:
        cp.wait()
    pltpu.sync_copy(vals_v, out_hbm.at[rows])


def _make_sc_gather(n_elems):
    assert n_elems % (_NW * _CH) == 0
    nch = n_elems // (_NW * _CH)
    assert nch % 8 == 0
    return functools.partial(
        pl.kernel,
        out_type=jax.ShapeDtypeStruct((n_elems // _CH, _CH), jnp.float32),
        mesh=plsc.VectorSubcoreMesh(core_axis_name="c", subcore_axis_name="s"),
        scratch_types=[
            pltpu.VMEM((nch, _CH), jnp.int32),
            pltpu.VMEM((nch, _CH), jnp.float32),
            pltpu.SemaphoreType.DMA,
        ],
    )(_sc_gather_body)


def _tc_dense_body(src_ref, sum_ref, lin_ref, acc_ref):
    i = pl.program_id(0)
    n_i = pl.num_programs(0)
    v = src_ref.shape[1]
    n_full = v // _LANES
    rem = v - n_full * _LANES

    p = jnp.clip(src_ref[...], _CLIP_LO, _CLIP_HI)
    for k in range(n_full):
        lin_ref[:, k, :] = p[:, k * _LANES:(k + 1) * _LANES]
    if rem:
        lin_ref[:, n_full, :rem] = p[:, n_full * _LANES:]

    dense = jnp.sum(jnp.log(1.0 - p))

    @pl.when(i == 0)
    def _():
        acc_ref[0] = 0.0

    acc_ref[0] += -dense

    @pl.when(i == n_i - 1)
    def _():
        sum_ref[0, 0] = acc_ref[0]


def _tc_dense(src, block_rows=256):
    b, v = src.shape
    kpad = (v + _LANES - 1) // _LANES
    grid = (b // block_rows,)
    return pl.pallas_call(
        _tc_dense_body,
        grid=grid,
        in_specs=[pl.BlockSpec((block_rows, v), lambda i: (i, 0))],
        out_specs=[
            pl.BlockSpec(memory_space=pltpu.SMEM),
            pl.BlockSpec((block_rows, kpad, _LANES), lambda i: (i, 0, 0)),
        ],
        out_shape=[
            jax.ShapeDtypeStruct((1, 1), jnp.float32),
            jax.ShapeDtypeStruct((b, kpad, _LANES), jnp.float32),
        ],
        scratch_shapes=[pltpu.SMEM((1,), jnp.float32)],
    )(src)


def _tc_corr_body(idx_ref, g_ref, sum_ref, out_ref):
    t = idx_ref.shape[0]
    tot = None
    for j in range(t):
        gj = g_ref[j]
        fj = jnp.log(1.0 - gj) - jnp.log(gj)
        if j == 0:
            tot = fj
        else:
            ij = idx_ref[j]
            dup = ij == idx_ref[0]
            for k in range(1, j):
                dup = dup | (ij == idx_ref[k])
            tot = tot + jnp.where(dup, 0.0, fj)
    out_ref[0, 0] = sum_ref[0, 0] + jnp.sum(tot)


def _tc_corr(idx_t, g_t, dense_sum):
    t, sub, lanes = idx_t.shape
    return pl.pallas_call(
        _tc_corr_body,
        in_specs=[
            pl.BlockSpec((t, sub, lanes), lambda: (0, 0, 0)),
            pl.BlockSpec((t, sub, lanes), lambda: (0, 0, 0)),
            pl.BlockSpec(memory_space=pltpu.SMEM),
        ],
        out_specs=pl.BlockSpec(memory_space=pltpu.SMEM),
        out_shape=jax.ShapeDtypeStruct((1, 1), jnp.float32),
    )(idx_t, g_t, dense_sum)


def kernel(src, tgt_indices):
    b, v = src.shape
    t = tgt_indices.shape[1]
    vpad = ((v + _LANES - 1) // _LANES) * _LANES
    idx32 = tgt_indices.astype(jnp.int32)

    tpad = -(-(b * t) // (_NW * _CH * 8)) * (_NW * _CH * 8) // b
    if tpad > t:
        idx32p = jnp.concatenate(
            [idx32] + [idx32[:, :1]] * (tpad - t), axis=1)
    else:
        idx32p = idx32

    flat_t = (idx32p + jnp.arange(b, dtype=jnp.int32)[:, None] * vpad).T
    idx_rows = flat_t.reshape(b * tpad // _CH, _CH)

    dense_sum, lin = _tc_dense(src)
    g_rows = _make_sc_gather(b * tpad)(lin.reshape(-1), idx_rows)

    idx_t = idx_rows.reshape(tpad, b // _LANES, _LANES)
    g_t = g_rows.reshape(tpad, b // _LANES, _LANES)
    total = _tc_corr(idx_t, g_t, dense_sum)
    scale = jnp.float32(1.0 / (b * v))
    return total[0, 0] * scale


# speedup vs baseline: 7.1334x; 1.4069x over previous
import functools

import jax
import jax.numpy as jnp
from jax import lax
from jax.experimental import pallas as pl
from jax.experimental.pallas import tpu as pltpu
from jax.experimental.pallas import tpu_sc as plsc

_NC = 2
_NS = 16
_NW = _NC * _NS
_CH = 128

_CLIP_LO = 1e-8
_CLIP_HI = 1.0 - 1e-8
_LANES = 128


def _sc_gather_body(src_hbm, idx_hbm, out_hbm, idx_v, vals_v, sem):
    wid = lax.axis_index("s") * _NC + lax.axis_index("c")
    nch = idx_v.shape[0]
    rows = pl.ds(wid * nch, nch)
    pltpu.sync_copy(idx_hbm.at[rows], idx_v)
    copies = []
    for c in range(nch):
        copies.append(
            pltpu.async_copy(src_hbm.at[idx_v.at[c]], vals_v.at[c], sem)
        )
    for cp in copies:
        cp.wait()
    pltpu.sync_copy(vals_v, out_hbm.at[rows])


def _make_sc_gather(n_elems):
    assert n_elems % (_NW * _CH) == 0
    nch = n_elems // (_NW * _CH)
    assert nch % 8 == 0
    return functools.partial(
        pl.kernel,
        out_type=jax.ShapeDtypeStruct((n_elems // _CH, _CH), jnp.float32),
        mesh=plsc.VectorSubcoreMesh(core_axis_name="c", subcore_axis_name="s"),
        scratch_types=[
            pltpu.VMEM((nch, _CH), jnp.int32),
            pltpu.VMEM((nch, _CH), jnp.float32),
            pltpu.SemaphoreType.DMA,
        ],
    )(_sc_gather_body)


def _tc_dense_body(src_ref, sum_ref, acc_ref):
    i = pl.program_id(0)
    n_i = pl.num_programs(0)

    p = jnp.clip(src_ref[...], _CLIP_LO, _CLIP_HI)
    dense = jnp.sum(jnp.log(1.0 - p))

    @pl.when(i == 0)
    def _():
        acc_ref[0] = 0.0

    acc_ref[0] += -dense

    @pl.when(i == n_i - 1)
    def _():
        sum_ref[0, 0] = acc_ref[0]


def _tc_dense(src_t, block_rows=200):
    v, b = src_t.shape
    grid = (v // block_rows,)
    return pl.pallas_call(
        _tc_dense_body,
        grid=grid,
        in_specs=[pl.BlockSpec((block_rows, b), lambda i: (i, 0))],
        out_specs=pl.BlockSpec(memory_space=pltpu.SMEM),
        out_shape=jax.ShapeDtypeStruct((1, 1), jnp.float32),
        scratch_shapes=[pltpu.SMEM((1,), jnp.float32)],
    )(src_t)


def _tc_corr_body(idx_ref, g_ref, sum_ref, out_ref):
    t = idx_ref.shape[0]
    tot = None
    for j in range(t):
        gj = jnp.clip(g_ref[j], _CLIP_LO, _CLIP_HI)
        fj = jnp.log(1.0 - gj) - jnp.log(gj)
        if j == 0:
            tot = fj
        else:
            ij = idx_ref[j]
            dup = ij == idx_ref[0]
            for k in range(1, j):
                dup = dup | (ij == idx_ref[k])
            tot = tot + jnp.where(dup, 0.0, fj)
    out_ref[0, 0] = sum_ref[0, 0] + jnp.sum(tot)


def _tc_corr(idx_t, g_t, dense_sum):
    t, sub, lanes = idx_t.shape
    return pl.pallas_call(
        _tc_corr_body,
        in_specs=[
            pl.BlockSpec((t, sub, lanes), lambda: (0, 0, 0)),
            pl.BlockSpec((t, sub, lanes), lambda: (0, 0, 0)),
            pl.BlockSpec(memory_space=pltpu.SMEM),
        ],
        out_specs=pl.BlockSpec(memory_space=pltpu.SMEM),
        out_shape=jax.ShapeDtypeStruct((1, 1), jnp.float32),
    )(idx_t, g_t, dense_sum)


def kernel(src, tgt_indices):
    b, v = src.shape
    t = tgt_indices.shape[1]
    idx32 = tgt_indices.astype(jnp.int32)

    tpad = -(-(b * t) // (_NW * _CH * 8)) * (_NW * _CH * 8) // b
    if tpad > t:
        idx32p = jnp.concatenate(
            [idx32] + [idx32[:, :1]] * (tpad - t), axis=1)
    else:
        idx32p = idx32

    flat_t = (idx32p * b + jnp.arange(b, dtype=jnp.int32)[:, None]).T
    idx_rows = flat_t.reshape(b * tpad // _CH, _CH)

    src_flat = src.T.reshape(-1)
    g_rows = _make_sc_gather(b * tpad)(src_flat, idx_rows)
    dense_sum = _tc_dense(src.T)

    idx_t = idx_rows.reshape(tpad, b // _LANES, _LANES)
    g_t = g_rows.reshape(tpad, b // _LANES, _LANES)
    total = _tc_corr(idx_t, g_t, dense_sum)
    scale = jnp.float32(1.0 / (b * v))
    return total[0, 0] * scale


# speedup vs baseline: 9.7575x; 1.3679x over previous
import functools

import jax
import jax.numpy as jnp
from jax import lax
from jax.experimental import pallas as pl
from jax.experimental.pallas import tpu as pltpu
from jax.experimental.pallas import tpu_sc as plsc

_NC = 2
_NS = 16
_NW = _NC * _NS
_CH = 128

_CLIP_LO = 1e-8
_CLIP_HI = 1.0 - 1e-8
_LANES = 128


def _sc_gather_body(src_hbm, idx_hbm, out_hbm, idx_v, vals_v, sem):
    wid = lax.axis_index("s") * _NC + lax.axis_index("c")
    nch = idx_v.shape[0]
    rows = pl.ds(wid * nch, nch)
    pltpu.sync_copy(idx_hbm.at[rows], idx_v)
    copies = []
    for c in range(nch):
        copies.append(
            pltpu.async_copy(src_hbm.at[idx_v.at[c]], vals_v.at[c], sem)
        )
    for cp in copies:
        cp.wait()
    pltpu.sync_copy(vals_v, out_hbm.at[rows])


def _make_sc_gather(n_elems):
    assert n_elems % (_NW * _CH) == 0
    nch = n_elems // (_NW * _CH)
    assert nch % 8 == 0
    return functools.partial(
        pl.kernel,
        out_type=jax.ShapeDtypeStruct((n_elems // _CH, _CH), jnp.float32),
        mesh=plsc.VectorSubcoreMesh(core_axis_name="c", subcore_axis_name="s"),
        scratch_types=[
            pltpu.VMEM((nch, _CH), jnp.int32),
            pltpu.VMEM((nch, _CH), jnp.float32),
            pltpu.SemaphoreType.DMA,
        ],
    )(_sc_gather_body)


def _tc_dense_body(src_ref, sum_ref, acc_ref):
    i = pl.program_id(0)
    n_i = pl.num_programs(0)

    p = jnp.clip(src_ref[...], _CLIP_LO, _CLIP_HI)
    dense = jnp.sum(jnp.log(1.0 - p))

    @pl.when(i == 0)
    def _():
        acc_ref[0] = 0.0

    acc_ref[0] += -dense

    @pl.when(i == n_i - 1)
    def _():
        sum_ref[0, 0] = acc_ref[0]


def _tc_dense(src_t, block_rows=200):
    v, b = src_t.shape
    grid = (v // block_rows,)
    return pl.pallas_call(
        _tc_dense_body,
        grid=grid,
        in_specs=[pl.BlockSpec((block_rows, b), lambda i: (i, 0))],
        out_specs=pl.BlockSpec(memory_space=pltpu.SMEM),
        out_shape=jax.ShapeDtypeStruct((1, 1), jnp.float32),
        scratch_shapes=[pltpu.SMEM((1,), jnp.float32)],
    )(src_t)


def _tc_corr_body(idx_ref, g_ref, sum_ref, out_ref):
    t = idx_ref.shape[0]
    tot = None
    for j in range(t):
        gj = jnp.clip(g_ref[j], _CLIP_LO, _CLIP_HI)
        fj = jnp.log(1.0 - gj) - jnp.log(gj)
        if j == 0:
            tot = fj
        else:
            ij = idx_ref[j]
            dup = ij == idx_ref[0]
            for k in range(1, j):
                dup = dup | (ij == idx_ref[k])
            tot = tot + jnp.where(dup, 0.0, fj)
    out_ref[0, 0] = sum_ref[0, 0] + jnp.sum(tot)


def _tc_corr(idx_t, g_t, dense_sum):
    t, sub, lanes = idx_t.shape
    return pl.pallas_call(
        _tc_corr_body,
        in_specs=[
            pl.BlockSpec((t, sub, lanes), lambda: (0, 0, 0)),
            pl.BlockSpec((t, sub, lanes), lambda: (0, 0, 0)),
            pl.BlockSpec(memory_space=pltpu.SMEM),
        ],
        out_specs=pl.BlockSpec(memory_space=pltpu.SMEM),
        out_shape=jax.ShapeDtypeStruct((1, 1), jnp.float32),
    )(idx_t, g_t, dense_sum)


def kernel(src, tgt_indices):
    b, v = src.shape
    t = tgt_indices.shape[1]
    idx32 = tgt_indices.astype(jnp.int32)

    tpad = -(-(b * t) // (_NW * _CH * 8)) * (_NW * _CH * 8) // b
    if tpad > t:
        idx32p = jnp.concatenate(
            [idx32] + [idx32[:, :1]] * (tpad - t), axis=1)
    else:
        idx32p = idx32

    r = jnp.arange(b, dtype=jnp.int32)[:, None]
    c = idx32p
    flat = (c // 8) * (8 * b) + (r // 128) * 1024 + (c % 8) * 128 + (r % 128)
    flat_t = flat.T
    idx_rows = flat_t.reshape(b * tpad // _CH, _CH)

    src_flat = (
        src.T.reshape(v // 8, 8, b // _LANES, _LANES)
        .transpose(0, 2, 1, 3)
        .reshape(-1)
    )
    g_rows = _make_sc_gather(b * tpad)(src_flat, idx_rows)
    dense_sum = _tc_dense(src.T)

    idx_t = idx_rows.reshape(tpad, b // _LANES, _LANES)
    g_t = g_rows.reshape(tpad, b // _LANES, _LANES)
    total = _tc_corr(idx_t, g_t, dense_sum)
    scale = jnp.float32(1.0 / (b * v))
    return total[0, 0] * scale
